# Initial kernel scaffold; baseline (speedup 1.0000x reference)
import jax
import jax.numpy as jnp
from jax.experimental import pallas as pl


def kernel(phi_indices, phi_values, phi_inverse_indices, phi_inverse_values, feature_indices, feature_values, weight_matrix, diagonal_weight_filter, dropout, device):
    raise NotImplementedError("write your pallas kernel here")


# speedup vs baseline: 3.7111x; 3.7111x over previous
import functools

import jax
import jax.numpy as jnp
from jax import lax
from jax.experimental import pallas as pl
from jax.experimental.pallas import tpu as pltpu
from jax.experimental.pallas import tpu_sc as plsc

L = 16
NC = 2
NS = 16
NW = NC * NS
K = 128
CH = 128


def _sc_spmm(n_pad, n_edges, use_diag):
    assert n_edges % (NW * K) == 0
    chunks = n_edges // (NW * K)
    e_per_w = n_edges // NW
    rpt = n_pad // NS
    assert n_pad % NS == 0 and rpt % K == 0 and rpt % 8 == 0

    mesh = plsc.VectorSubcoreMesh(core_axis_name="c", subcore_axis_name="s")

    scratch = [
        pltpu.VMEM((K,), jnp.int32),
        pltpu.VMEM((K,), jnp.int32),
        pltpu.VMEM((K,), jnp.float32),
        pltpu.VMEM((K, CH), jnp.float32),
        pltpu.VMEM_SHARED((n_pad, CH), jnp.float32),
        pltpu.SemaphoreType.DMA,
    ]
    if use_diag:
        scratch.append(pltpu.VMEM((n_pad,), jnp.float32))

    def body(*refs):
        if use_diag:
            (table, cols, rows, vals, diag, out,
             cidx, ridx, vv, rbuf, acc, sem, diag_v) = refs
        else:
            (table, cols, rows, vals, out,
             cidx, ridx, vv, rbuf, acc, sem) = refs
        c = lax.axis_index("c")
        s = lax.axis_index("s")
        w = s * NC + c

        def zero_row(i, _):
            for u in range(CH // L):
                rbuf[i, pl.ds(u * L, L)] = jnp.zeros((L,), jnp.float32)
            return 0
        lax.fori_loop(0, K, zero_row, 0)
        for t in range(rpt // K):
            pltpu.sync_copy(rbuf, acc.at[pl.ds(s * rpt + t * K, K)])
        if use_diag:
            pltpu.sync_copy(diag, diag_v)
        plsc.subcore_barrier()

        def chunk(j, _):
            base = w * e_per_w + j * K
            pltpu.sync_copy(cols.at[pl.ds(base, K)], cidx)
            pltpu.sync_copy(rows.at[pl.ds(base, K)], ridx)
            pltpu.sync_copy(vals.at[pl.ds(base, K)], vv)
            if use_diag:
                for g in range(K // L):
                    sl = pl.ds(g * L, L)
                    vv[sl] = vv[sl] * plsc.load_gather(diag_v, [cidx[sl]])
            pltpu.async_copy(table.at[cidx], rbuf, sem).wait()

            def scale(i, _):
                vvec = plsc.load_gather(vv, [jnp.broadcast_to(i, (L,))])
                for u in range(CH // L):
                    sl = pl.ds(u * L, L)
                    rbuf[i, sl] = rbuf[i, sl] * vvec
                return 0
            lax.fori_loop(0, K, scale, 0)
            pltpu.sync_copy(rbuf, acc.at[ridx], add=True)
            return 0
        lax.fori_loop(0, chunks, chunk, 0)

        plsc.subcore_barrier()
        pltpu.sync_copy(acc.at[pl.ds(s * rpt, rpt)],
                        out.at[c, pl.ds(s * rpt, rpt)])

    return pl.kernel(
        body,
        out_type=jax.ShapeDtypeStruct((NC, n_pad, CH), jnp.float32),
        mesh=mesh,
        scratch_types=scratch,
        compiler_params=pltpu.CompilerParams(needs_layout_passes=False),
    )


def _combine(parts, relu):
    n_pad = parts.shape[1]
    br = 512
    assert n_pad % br == 0

    def body(p_ref, o_ref):
        x = p_ref[0] + p_ref[1]
        if relu:
            x = jnp.maximum(x, 0.0)
        o_ref[...] = x

    return pl.pallas_call(
        body,
        grid=(n_pad // br,),
        in_specs=[pl.BlockSpec((2, br, CH), lambda i: (0, i, 0))],
        out_specs=pl.BlockSpec((br, CH), lambda i: (i, 0)),
        out_shape=jax.ShapeDtypeStruct((n_pad, CH), jnp.float32),
    )(parts)


def _pad_edges(cols, rows, vals, dummy_row):
    n = cols.shape[0]
    tgt = ((n + NW * K - 1) // (NW * K)) * (NW * K)
    pad = tgt - n
    if pad:
        cols = jnp.pad(cols, (0, pad))
        rows = jnp.pad(rows, (0, pad), constant_values=dummy_row)
        vals = jnp.pad(vals, (0, pad))
    return cols, rows, vals


def kernel(phi_indices, phi_values, phi_inverse_indices, phi_inverse_values,
           feature_indices, feature_values, weight_matrix,
           diagonal_weight_filter, dropout=0, device=0):
    n = diagonal_weight_filter.shape[0]
    n_pad = ((n + NS * K - 1) // (NS * K)) * (NS * K)

    fc, fr, fv = _pad_edges(feature_indices[1].astype(jnp.int32),
                            feature_indices[0].astype(jnp.int32),
                            feature_values, n)
    pa = _sc_spmm(n_pad, fc.shape[0], False)(weight_matrix, fc, fr, fv)
    filtered = _combine(pa, relu=False)

    ic = phi_inverse_indices[1].astype(jnp.int32)
    ir = phi_inverse_indices[0].astype(jnp.int32)
    ic, ir, iv = _pad_edges(ic, ir, phi_inverse_values, n)
    pb = _sc_spmm(n_pad, ic.shape[0], False)(filtered, ic, ir, iv)
    y1 = _combine(pb, relu=False)

    diag_pad = jnp.pad(diagonal_weight_filter.reshape(-1), (0, n_pad - n))
    pc_ = phi_indices[1].astype(jnp.int32)
    pr = phi_indices[0].astype(jnp.int32)
    pc_, pr, pv = _pad_edges(pc_, pr, phi_values, n)
    pcparts = _sc_spmm(n_pad, pc_.shape[0], True)(y1, pc_, pr, pv, diag_pad)
    out = _combine(pcparts, relu=True)
    return out[:n]
